# baseline (speedup 1.0000x reference)
import functools

import jax
import jax.numpy as jnp
from jax import lax
from jax.experimental import pallas as pl
from jax.experimental.pallas import tpu as pltpu
from jax.experimental.pallas import tpu_sc as plsc

N = 10000
E = 320000
D_IN = 128
HID = 128
LAT = 64
G = 64

DEXT = 144
NC = 2
NS = 16
NW = NC * NS
EPW = E // NW
CH = 80
NCHUNK = EPW // CH
ROWS_PER_TILE = N // NS


def _gat_dense_kernel(x_ref, w_ref, a_ref, hext_ref, ad_ref):
    h = jnp.dot(x_ref[...], w_ref[...], preferred_element_type=jnp.float32)
    al = jnp.dot(h, a_ref[...], preferred_element_type=jnp.float32)
    nrow = h.shape[0]
    hext_ref[...] = jnp.concatenate(
        [h,
         jnp.ones((nrow, 1), jnp.float32),
         al[:, 0:1],
         jnp.zeros((nrow, DEXT - HID - 2), jnp.float32)], axis=1)
    ad_ref[...] = al[:, 1:2]


def _gat_norm_dense_kernel(acc_ref, b_ref, w_ref, a_ref, hext_ref, ad_ref):
    accsum = acc_ref[0] + acc_ref[1]
    denom = accsum[:, HID:HID + 1]
    g = jax.nn.relu(accsum[:, :HID] / (denom + 1e-16) + b_ref[...])
    h = jnp.dot(g, w_ref[...], preferred_element_type=jnp.float32)
    al = jnp.dot(h, a_ref[...], preferred_element_type=jnp.float32)
    nrow = h.shape[0]
    hext_ref[...] = jnp.concatenate(
        [h,
         jnp.ones((nrow, 1), jnp.float32),
         al[:, 0:1],
         jnp.zeros((nrow, DEXT - HID - 2), jnp.float32)], axis=1)
    ad_ref[...] = al[:, 1:2]


def _finalize_kernel(acc_ref, b_ref, batch_ref, fcw_ref, fcb_ref, z_ref):
    accsum = acc_ref[0] + acc_ref[1]
    denom = accsum[:, HID:HID + 1]
    g = jax.nn.relu(accsum[:, :HID] / (denom + 1e-16) + b_ref[...])
    gid = lax.broadcasted_iota(jnp.int32, (G, N), 0)
    oh = (gid == batch_ref[...]).astype(jnp.float32)
    pooled = jnp.dot(oh, g, preferred_element_type=jnp.float32)
    counts = jnp.sum(oh, axis=1, keepdims=True)
    pooled = pooled / (counts + 1e-16)
    z_ref[...] = jnp.dot(pooled, fcw_ref[...],
                         preferred_element_type=jnp.float32) + fcb_ref[...]


NRB = 3
NIB = 5


def _edge_body(hext_hbm, ad_hbm, edges_hbm, zeros_hbm, out_hbm,
               acc_s, eidx, adb, rows, wbuf, isems, gsems, asems, ssems):
    c = lax.axis_index("c")
    s = lax.axis_index("s")
    wid = c * NS + s
    rbase = wid * NCHUNK

    pltpu.sync_copy(zeros_hbm.at[pl.ds(s * ROWS_PER_TILE, ROWS_PER_TILE)],
                    acc_s.at[pl.ds(s * ROWS_PER_TILE, ROWS_PER_TILE)])
    plsc.subcore_barrier()

    def start_idx(j):
        sl = j % NIB
        pltpu.async_copy(edges_hbm.at[rbase + j], eidx.at[sl], isems.at[sl])

    def wait_idx(j):
        sl = j % NIB
        pltpu.make_async_copy(edges_hbm.at[rbase + j], eidx.at[sl],
                              isems.at[sl]).wait()

    def start_gather(j):
        sl, rb = j % NIB, j % NRB
        pltpu.async_copy(hext_hbm.at[eidx.at[sl, 0]], rows.at[rb], gsems.at[rb])
        pltpu.async_copy(ad_hbm.at[eidx.at[sl, 1]], adb.at[rb], asems.at[rb])

    def wait_gather(j):
        sl, rb = j % NIB, j % NRB
        pltpu.make_async_copy(hext_hbm.at[eidx.at[sl, 0]], rows.at[rb],
                              gsems.at[rb]).wait()
        pltpu.make_async_copy(ad_hbm.at[eidx.at[sl, 1]], adb.at[rb],
                              asems.at[rb]).wait()

    def start_scatter(j):
        sl, rb = j % NIB, j % NRB
        pltpu.async_copy(rows.at[rb], acc_s.at[eidx.at[sl, 1]], ssems.at[rb],
                         add=True)

    def wait_scatter(j):
        sl, rb = j % NIB, j % NRB
        pltpu.make_async_copy(rows.at[rb], acc_s.at[eidx.at[sl, 1]],
                              ssems.at[rb]).wait()

    def compute(j):
        rb = j % NRB
        for gi in range(CH // 16):
            rowi = lax.iota(jnp.int32, 16) + gi * 16
            asv = plsc.load_gather(rows.at[rb],
                                   [rowi, jnp.full((16,), HID + 1, jnp.int32)])
            adv = adb[rb, pl.ds(gi * 16, 16)]
            e = asv + adv
            e = jnp.where(e >= 0.0, e, 0.2 * e)
            wbuf[pl.ds(gi * 16, 16)] = jnp.exp(e)

        def scale(i, carry2):
            wbc = plsc.load_gather(wbuf, [jnp.full((16,), 0, jnp.int32) + i])
            for f in range(HID // 16):
                rows[rb, i, pl.ds(f * 16, 16)] = (
                    rows[rb, i, pl.ds(f * 16, 16)] * wbc)
            rows[rb, i, pl.ds(HID, 16)] = wbc
            return carry2
        lax.fori_loop(0, CH, scale, 0, unroll=2)

    start_idx(0)
    start_idx(1)
    wait_idx(0); start_gather(0); start_idx(2)
    wait_idx(1); start_gather(1); start_idx(3)
    wait_gather(0); compute(0); start_scatter(0)
    wait_idx(2); start_gather(2); start_idx(4)
    wait_gather(1); compute(1); start_scatter(1)

    def steady(q, carry):
        for k in range(15):
            j = 2 + q * 15 + k
            wait_scatter(j - 2)
            wait_idx(j + 1)
            start_gather(j + 1)
            start_idx(j + 3)
            wait_gather(j)
            compute(j)
            start_scatter(j)
        return carry
    lax.fori_loop(0, (NCHUNK - 5) // 15, steady, 0)

    for j in range(NCHUNK - 3, NCHUNK):
        wait_scatter(j - 2)
        if j + 1 < NCHUNK:
            wait_idx(j + 1)
            start_gather(j + 1)
        wait_gather(j)
        compute(j)
        start_scatter(j)
    wait_scatter(NCHUNK - 2)
    wait_scatter(NCHUNK - 1)

    plsc.subcore_barrier()
    pltpu.sync_copy(acc_s.at[pl.ds(s * ROWS_PER_TILE, ROWS_PER_TILE)],
                    out_hbm.at[c, pl.ds(s * ROWS_PER_TILE, ROWS_PER_TILE)])


_edge_pass = pl.kernel(
    _edge_body,
    out_type=jax.ShapeDtypeStruct((NC, N, DEXT), jnp.float32),
    mesh=plsc.VectorSubcoreMesh(core_axis_name="c", subcore_axis_name="s"),
    scratch_types=[
        pltpu.VMEM_SHARED((N, DEXT), jnp.float32),
        pltpu.VMEM((NIB, 2, CH), jnp.int32),
        pltpu.VMEM((NRB, CH), jnp.float32),
        pltpu.VMEM((NRB, CH, DEXT), jnp.float32),
        pltpu.VMEM((CH,), jnp.float32),
        pltpu.SemaphoreType.DMA((NIB,)),
        pltpu.SemaphoreType.DMA((NRB,)),
        pltpu.SemaphoreType.DMA((NRB,)),
        pltpu.SemaphoreType.DMA((NRB,)),
    ],
    compiler_params=pltpu.CompilerParams(use_tc_tiling_on_sc=False,
                                         needs_layout_passes=False),
)

_BLK = 2000


def _dense1(x, W, a2):
    return pl.pallas_call(
        _gat_dense_kernel,
        grid=(N // _BLK,),
        in_specs=[
            pl.BlockSpec((_BLK, D_IN), lambda i: (i, 0)),
            pl.BlockSpec((D_IN, HID), lambda i: (0, 0)),
            pl.BlockSpec((HID, 2), lambda i: (0, 0)),
        ],
        out_specs=[
            pl.BlockSpec((_BLK, DEXT), lambda i: (i, 0)),
            pl.BlockSpec((_BLK, 1), lambda i: (i, 0)),
        ],
        out_shape=[
            jax.ShapeDtypeStruct((N, DEXT), jnp.float32),
            jax.ShapeDtypeStruct((N, 1), jnp.float32),
        ],
    )(x, W, a2)


def _dense2(acc, b, W, a2):
    return pl.pallas_call(
        _gat_norm_dense_kernel,
        grid=(N // _BLK,),
        in_specs=[
            pl.BlockSpec((NC, _BLK, DEXT), lambda i: (0, i, 0)),
            pl.BlockSpec((1, HID), lambda i: (0, 0)),
            pl.BlockSpec((HID, HID), lambda i: (0, 0)),
            pl.BlockSpec((HID, 2), lambda i: (0, 0)),
        ],
        out_specs=[
            pl.BlockSpec((_BLK, DEXT), lambda i: (i, 0)),
            pl.BlockSpec((_BLK, 1), lambda i: (i, 0)),
        ],
        out_shape=[
            jax.ShapeDtypeStruct((N, DEXT), jnp.float32),
            jax.ShapeDtypeStruct((N, 1), jnp.float32),
        ],
    )(acc, b, W, a2)


def _finalize(acc, b, batch_row, fc_w, fc_b):
    return pl.pallas_call(
        _finalize_kernel,
        out_shape=jax.ShapeDtypeStruct((G, LAT), jnp.float32),
    )(acc, b, batch_row, fc_w, fc_b)


@jax.jit
def kernel(x, edge_index, batch, W1, a1_src, a1_dst, b1,
           W2, a2_src, a2_dst, b2, fc_w, fc_b):
    edges = jnp.transpose(edge_index.reshape(2, NW * NCHUNK, CH), (1, 0, 2))
    zeros = jnp.zeros((N, DEXT), jnp.float32)

    a1 = jnp.stack([a1_src, a1_dst], axis=1)
    hext1, ad1 = _dense1(x, W1, a1)
    acc1 = _edge_pass(hext1, ad1.reshape(N), edges, zeros)

    a2 = jnp.stack([a2_src, a2_dst], axis=1)
    hext2, ad2 = _dense2(acc1, b1.reshape(1, HID), W2, a2)
    acc2 = _edge_pass(hext2, ad2.reshape(N), edges, zeros)

    return _finalize(acc2, b2.reshape(1, HID), batch.reshape(1, N),
                     fc_w, fc_b.reshape(1, LAT))


# speedup vs baseline: 1.0223x; 1.0223x over previous
import functools

import jax
import jax.numpy as jnp
from jax import lax
from jax.experimental import pallas as pl
from jax.experimental.pallas import tpu as pltpu
from jax.experimental.pallas import tpu_sc as plsc

N = 10000
E = 320000
D_IN = 128
HID = 128
LAT = 64
G = 64

DEXT = 144
NC = 2
NS = 16
NW = NC * NS
EPW = E // NW
CH = 80
NCHUNK = EPW // CH
ROWS_PER_TILE = N // NS


def _gat_dense_kernel(x_ref, w_ref, a_ref, hext_ref, ad_ref):
    h = jnp.dot(x_ref[...], w_ref[...], preferred_element_type=jnp.float32)
    al = jnp.dot(h, a_ref[...], preferred_element_type=jnp.float32)
    nrow = h.shape[0]
    hext_ref[...] = jnp.concatenate(
        [h,
         jnp.ones((nrow, 1), jnp.float32),
         al[:, 0:1],
         jnp.zeros((nrow, DEXT - HID - 2), jnp.float32)], axis=1)
    ad_ref[...] = al[:, 1:2]


def _gat_norm_dense_kernel(acc_ref, b_ref, w_ref, a_ref, hext_ref, ad_ref):
    accsum = acc_ref[0] + acc_ref[1]
    denom = accsum[:, HID:HID + 1]
    g = jax.nn.relu(accsum[:, :HID] / (denom + 1e-16) + b_ref[...])
    h = jnp.dot(g, w_ref[...], preferred_element_type=jnp.float32)
    al = jnp.dot(h, a_ref[...], preferred_element_type=jnp.float32)
    nrow = h.shape[0]
    hext_ref[...] = jnp.concatenate(
        [h,
         jnp.ones((nrow, 1), jnp.float32),
         al[:, 0:1],
         jnp.zeros((nrow, DEXT - HID - 2), jnp.float32)], axis=1)
    ad_ref[...] = al[:, 1:2]


def _finalize_kernel(acc_ref, b_ref, batch_ref, fcw_ref, fcb_ref, z_ref):
    accsum = acc_ref[0] + acc_ref[1]
    denom = accsum[:, HID:HID + 1]
    g = jax.nn.relu(accsum[:, :HID] / (denom + 1e-16) + b_ref[...])
    gid = lax.broadcasted_iota(jnp.int32, (G, N), 0)
    oh = (gid == batch_ref[...]).astype(jnp.float32)
    pooled = jnp.dot(oh, g, preferred_element_type=jnp.float32)
    counts = jnp.sum(oh, axis=1, keepdims=True)
    pooled = pooled / (counts + 1e-16)
    z_ref[...] = jnp.dot(pooled, fcw_ref[...],
                         preferred_element_type=jnp.float32) + fcb_ref[...]


NRB = 3
NIB = 5


def _edge_body(hext_hbm, ad_hbm, src_hbm, dst_hbm, zeros_hbm, out_hbm,
               acc_s, sidx, didx, adb, rows, wbuf, isems, gsems, asems, ssems):
    c = lax.axis_index("c")
    s = lax.axis_index("s")
    wid = c * NS + s
    rbase = wid * NCHUNK

    pltpu.sync_copy(zeros_hbm.at[pl.ds(s * ROWS_PER_TILE, ROWS_PER_TILE)],
                    acc_s.at[pl.ds(s * ROWS_PER_TILE, ROWS_PER_TILE)])
    plsc.subcore_barrier()

    def start_idx(j):
        sl = j % NIB
        pltpu.async_copy(src_hbm.at[rbase + j], sidx.at[sl], isems.at[sl])
        pltpu.async_copy(dst_hbm.at[rbase + j], didx.at[sl], isems.at[sl])

    def wait_idx(j):
        sl = j % NIB
        pltpu.make_async_copy(src_hbm.at[rbase + j], sidx.at[sl],
                              isems.at[sl]).wait()
        pltpu.make_async_copy(dst_hbm.at[rbase + j], didx.at[sl],
                              isems.at[sl]).wait()

    def start_gather(j):
        sl, rb = j % NIB, j % NRB
        pltpu.async_copy(hext_hbm.at[sidx.at[sl]], rows.at[rb], gsems.at[rb])
        pltpu.async_copy(ad_hbm.at[didx.at[sl]], adb.at[rb], asems.at[rb])

    def wait_gather(j):
        sl, rb = j % NIB, j % NRB
        pltpu.make_async_copy(hext_hbm.at[sidx.at[sl]], rows.at[rb],
                              gsems.at[rb]).wait()
        pltpu.make_async_copy(ad_hbm.at[didx.at[sl]], adb.at[rb],
                              asems.at[rb]).wait()

    def start_scatter(j):
        sl, rb = j % NIB, j % NRB
        pltpu.async_copy(rows.at[rb], acc_s.at[didx.at[sl]], ssems.at[rb],
                         add=True)

    def wait_scatter(j):
        sl, rb = j % NIB, j % NRB
        pltpu.make_async_copy(rows.at[rb], acc_s.at[didx.at[sl]],
                              ssems.at[rb]).wait()

    def compute(j):
        rb = j % NRB
        for gi in range(CH // 16):
            rowi = lax.iota(jnp.int32, 16) + gi * 16
            asv = plsc.load_gather(rows.at[rb],
                                   [rowi, jnp.full((16,), HID + 1, jnp.int32)])
            adv = adb[rb, pl.ds(gi * 16, 16)]
            e = asv + adv
            e = jnp.where(e >= 0.0, e, 0.2 * e)
            wbuf[pl.ds(gi * 16, 16)] = jnp.exp(e)

        def scale(i, carry2):
            wbc = plsc.load_gather(wbuf, [jnp.full((16,), 0, jnp.int32) + i])
            for f in range(HID // 16):
                rows[rb, i, pl.ds(f * 16, 16)] = (
                    rows[rb, i, pl.ds(f * 16, 16)] * wbc)
            rows[rb, i, pl.ds(HID, 16)] = wbc
            return carry2
        lax.fori_loop(0, CH, scale, 0, unroll=4)

    start_idx(0)
    start_idx(1)
    wait_idx(0); start_gather(0); start_idx(2)
    wait_idx(1); start_gather(1); start_idx(3)
    wait_gather(0); compute(0); start_scatter(0)
    wait_idx(2); start_gather(2); start_idx(4)
    wait_gather(1); compute(1); start_scatter(1)

    def steady(q, carry):
        for k in range(15):
            j = 2 + q * 15 + k
            wait_scatter(j - 2)
            wait_idx(j + 1)
            start_gather(j + 1)
            start_idx(j + 3)
            wait_gather(j)
            compute(j)
            start_scatter(j)
        return carry
    lax.fori_loop(0, (NCHUNK - 5) // 15, steady, 0)

    for j in range(NCHUNK - 3, NCHUNK):
        wait_scatter(j - 2)
        if j + 1 < NCHUNK:
            wait_idx(j + 1)
            start_gather(j + 1)
        wait_gather(j)
        compute(j)
        start_scatter(j)
    wait_scatter(NCHUNK - 2)
    wait_scatter(NCHUNK - 1)

    plsc.subcore_barrier()
    pltpu.sync_copy(acc_s.at[pl.ds(s * ROWS_PER_TILE, ROWS_PER_TILE)],
                    out_hbm.at[c, pl.ds(s * ROWS_PER_TILE, ROWS_PER_TILE)])


_edge_pass = pl.kernel(
    _edge_body,
    out_type=jax.ShapeDtypeStruct((NC, N, DEXT), jnp.float32),
    mesh=plsc.VectorSubcoreMesh(core_axis_name="c", subcore_axis_name="s"),
    scratch_types=[
        pltpu.VMEM_SHARED((N, DEXT), jnp.float32),
        pltpu.VMEM((NIB, CH), jnp.int32),
        pltpu.VMEM((NIB, CH), jnp.int32),
        pltpu.VMEM((NRB, CH), jnp.float32),
        pltpu.VMEM((NRB, CH, DEXT), jnp.float32),
        pltpu.VMEM((CH,), jnp.float32),
        pltpu.SemaphoreType.DMA((NIB,)),
        pltpu.SemaphoreType.DMA((NRB,)),
        pltpu.SemaphoreType.DMA((NRB,)),
        pltpu.SemaphoreType.DMA((NRB,)),
    ],
    compiler_params=pltpu.CompilerParams(use_tc_tiling_on_sc=False,
                                         needs_layout_passes=False),
)

_BLK = 2000


def _dense1(x, W, a2):
    return pl.pallas_call(
        _gat_dense_kernel,
        grid=(N // _BLK,),
        in_specs=[
            pl.BlockSpec((_BLK, D_IN), lambda i: (i, 0)),
            pl.BlockSpec((D_IN, HID), lambda i: (0, 0)),
            pl.BlockSpec((HID, 2), lambda i: (0, 0)),
        ],
        out_specs=[
            pl.BlockSpec((_BLK, DEXT), lambda i: (i, 0)),
            pl.BlockSpec((_BLK, 1), lambda i: (i, 0)),
        ],
        out_shape=[
            jax.ShapeDtypeStruct((N, DEXT), jnp.float32),
            jax.ShapeDtypeStruct((N, 1), jnp.float32),
        ],
    )(x, W, a2)


def _dense2(acc, b, W, a2):
    return pl.pallas_call(
        _gat_norm_dense_kernel,
        grid=(N // _BLK,),
        in_specs=[
            pl.BlockSpec((NC, _BLK, DEXT), lambda i: (0, i, 0)),
            pl.BlockSpec((1, HID), lambda i: (0, 0)),
            pl.BlockSpec((HID, HID), lambda i: (0, 0)),
            pl.BlockSpec((HID, 2), lambda i: (0, 0)),
        ],
        out_specs=[
            pl.BlockSpec((_BLK, DEXT), lambda i: (i, 0)),
            pl.BlockSpec((_BLK, 1), lambda i: (i, 0)),
        ],
        out_shape=[
            jax.ShapeDtypeStruct((N, DEXT), jnp.float32),
            jax.ShapeDtypeStruct((N, 1), jnp.float32),
        ],
    )(acc, b, W, a2)


def _finalize(acc, b, batch_row, fc_w, fc_b):
    return pl.pallas_call(
        _finalize_kernel,
        out_shape=jax.ShapeDtypeStruct((G, LAT), jnp.float32),
    )(acc, b, batch_row, fc_w, fc_b)


@jax.jit
def kernel(x, edge_index, batch, W1, a1_src, a1_dst, b1,
           W2, a2_src, a2_dst, b2, fc_w, fc_b):
    src = edge_index[0].reshape(NW * NCHUNK, CH)
    dst = edge_index[1].reshape(NW * NCHUNK, CH)
    zeros = jnp.zeros((N, DEXT), jnp.float32)

    a1 = jnp.stack([a1_src, a1_dst], axis=1)
    hext1, ad1 = _dense1(x, W1, a1)
    acc1 = _edge_pass(hext1, ad1.reshape(N), src, dst, zeros)

    a2 = jnp.stack([a2_src, a2_dst], axis=1)
    hext2, ad2 = _dense2(acc1, b1.reshape(1, HID), W2, a2)
    acc2 = _edge_pass(hext2, ad2.reshape(N), src, dst, zeros)

    return _finalize(acc2, b2.reshape(1, HID), batch.reshape(1, N),
                     fc_w, fc_b.reshape(1, LAT))


# speedup vs baseline: 1.0290x; 1.0066x over previous
import functools

import jax
import jax.numpy as jnp
from jax import lax
from jax.experimental import pallas as pl
from jax.experimental.pallas import tpu as pltpu
from jax.experimental.pallas import tpu_sc as plsc

N = 10000
E = 320000
D_IN = 128
HID = 128
LAT = 64
G = 64

DEXT = 144
NC = 2
NS = 16
NW = NC * NS
EPW = E // NW
CH = 80
NCHUNK = EPW // CH
ROWS_PER_TILE = N // NS


def _gat_dense_kernel(x_ref, w_ref, a_ref, hext_ref, ad_ref):
    h = jnp.dot(x_ref[...], w_ref[...], preferred_element_type=jnp.float32)
    al = jnp.dot(h, a_ref[...], preferred_element_type=jnp.float32)
    nrow = h.shape[0]
    hext_ref[...] = jnp.concatenate(
        [h,
         jnp.ones((nrow, 1), jnp.float32),
         al[:, 0:1],
         jnp.zeros((nrow, DEXT - HID - 2), jnp.float32)], axis=1)
    ad_ref[...] = al[:, 1:2]


def _gat_norm_dense_kernel(acc_ref, b_ref, w_ref, a_ref, hext_ref, ad_ref):
    accsum = acc_ref[0] + acc_ref[1]
    denom = accsum[:, HID:HID + 1]
    g = jax.nn.relu(accsum[:, :HID] / (denom + 1e-16) + b_ref[...])
    h = jnp.dot(g, w_ref[...], preferred_element_type=jnp.float32)
    al = jnp.dot(h, a_ref[...], preferred_element_type=jnp.float32)
    nrow = h.shape[0]
    hext_ref[...] = jnp.concatenate(
        [h,
         jnp.ones((nrow, 1), jnp.float32),
         al[:, 0:1],
         jnp.zeros((nrow, DEXT - HID - 2), jnp.float32)], axis=1)
    ad_ref[...] = al[:, 1:2]


def _finalize_kernel(acc_ref, b_ref, batch_ref, fcw_ref, fcb_ref, z_ref):
    accsum = acc_ref[0] + acc_ref[1]
    denom = accsum[:, HID:HID + 1]
    g = jax.nn.relu(accsum[:, :HID] / (denom + 1e-16) + b_ref[...])
    gid = lax.broadcasted_iota(jnp.int32, (G, N), 0)
    oh = (gid == batch_ref[...]).astype(jnp.float32)
    pooled = jnp.dot(oh, g, preferred_element_type=jnp.float32)
    counts = jnp.sum(oh, axis=1, keepdims=True)
    pooled = pooled / (counts + 1e-16)
    z_ref[...] = jnp.dot(pooled, fcw_ref[...],
                         preferred_element_type=jnp.float32) + fcb_ref[...]


NRB = 3
NIB = 5


def _edge_body(hext_hbm, ad_hbm, src_hbm, dst_hbm, zeros_hbm, out_hbm,
               acc_s, sidx, didx, adb, rows, wbuf, isems, gsems, asems, ssems):
    c = lax.axis_index("c")
    s = lax.axis_index("s")
    wid = c * NS + s
    rbase = wid * NCHUNK

    def start_idx(j):
        sl = j % NIB
        pltpu.async_copy(src_hbm.at[rbase + j], sidx.at[sl], isems.at[sl])
        pltpu.async_copy(dst_hbm.at[rbase + j], didx.at[sl], isems.at[sl])

    def wait_idx(j):
        sl = j % NIB
        pltpu.make_async_copy(src_hbm.at[rbase + j], sidx.at[sl],
                              isems.at[sl]).wait()
        pltpu.make_async_copy(dst_hbm.at[rbase + j], didx.at[sl],
                              isems.at[sl]).wait()

    def start_gather(j):
        sl, rb = j % NIB, j % NRB
        pltpu.async_copy(hext_hbm.at[sidx.at[sl]], rows.at[rb], gsems.at[rb])
        pltpu.async_copy(ad_hbm.at[didx.at[sl]], adb.at[rb], asems.at[rb])

    def wait_gather(j):
        sl, rb = j % NIB, j % NRB
        pltpu.make_async_copy(hext_hbm.at[sidx.at[sl]], rows.at[rb],
                              gsems.at[rb]).wait()
        pltpu.make_async_copy(ad_hbm.at[didx.at[sl]], adb.at[rb],
                              asems.at[rb]).wait()

    def start_scatter(j):
        sl, rb = j % NIB, j % NRB
        pltpu.async_copy(rows.at[rb], acc_s.at[didx.at[sl]], ssems.at[rb],
                         add=True)

    def wait_scatter(j):
        sl, rb = j % NIB, j % NRB
        pltpu.make_async_copy(rows.at[rb], acc_s.at[didx.at[sl]],
                              ssems.at[rb]).wait()

    def compute(j):
        rb = j % NRB
        for gi in range(CH // 16):
            rowi = lax.iota(jnp.int32, 16) + gi * 16
            asv = plsc.load_gather(rows.at[rb],
                                   [rowi, jnp.full((16,), HID + 1, jnp.int32)])
            adv = adb[rb, pl.ds(gi * 16, 16)]
            e = asv + adv
            e = jnp.where(e >= 0.0, e, 0.2 * e)
            wbuf[pl.ds(gi * 16, 16)] = jnp.exp(e)

        def scale(i, carry2):
            wbc = plsc.load_gather(wbuf, [jnp.full((16,), 0, jnp.int32) + i])
            for f in range(HID // 16):
                rows[rb, i, pl.ds(f * 16, 16)] = (
                    rows[rb, i, pl.ds(f * 16, 16)] * wbc)
            rows[rb, i, pl.ds(HID, 16)] = wbc
            return carry2
        lax.fori_loop(0, CH, scale, 0, unroll=4)

    start_idx(0)
    start_idx(1)
    wait_idx(0); start_gather(0); start_idx(2)
    wait_idx(1); start_gather(1); start_idx(3)
    pltpu.sync_copy(zeros_hbm.at[pl.ds(s * ROWS_PER_TILE, ROWS_PER_TILE)],
                    acc_s.at[pl.ds(s * ROWS_PER_TILE, ROWS_PER_TILE)])
    plsc.subcore_barrier()
    wait_gather(0); compute(0); start_scatter(0)
    wait_idx(2); start_gather(2); start_idx(4)
    wait_gather(1); compute(1); start_scatter(1)

    def steady(q, carry):
        for k in range(15):
            j = 2 + q * 15 + k
            wait_scatter(j - 2)
            wait_idx(j + 1)
            start_gather(j + 1)
            start_idx(j + 3)
            wait_gather(j)
            compute(j)
            start_scatter(j)
        return carry
    lax.fori_loop(0, (NCHUNK - 5) // 15, steady, 0)

    for j in range(NCHUNK - 3, NCHUNK):
        wait_scatter(j - 2)
        if j + 1 < NCHUNK:
            wait_idx(j + 1)
            start_gather(j + 1)
        wait_gather(j)
        compute(j)
        start_scatter(j)
    wait_scatter(NCHUNK - 2)
    wait_scatter(NCHUNK - 1)

    plsc.subcore_barrier()
    pltpu.sync_copy(acc_s.at[pl.ds(s * ROWS_PER_TILE, ROWS_PER_TILE)],
                    out_hbm.at[c, pl.ds(s * ROWS_PER_TILE, ROWS_PER_TILE)])


_edge_pass = pl.kernel(
    _edge_body,
    out_type=jax.ShapeDtypeStruct((NC, N, DEXT), jnp.float32),
    mesh=plsc.VectorSubcoreMesh(core_axis_name="c", subcore_axis_name="s"),
    scratch_types=[
        pltpu.VMEM_SHARED((N, DEXT), jnp.float32),
        pltpu.VMEM((NIB, CH), jnp.int32),
        pltpu.VMEM((NIB, CH), jnp.int32),
        pltpu.VMEM((NRB, CH), jnp.float32),
        pltpu.VMEM((NRB, CH, DEXT), jnp.float32),
        pltpu.VMEM((CH,), jnp.float32),
        pltpu.SemaphoreType.DMA((NIB,)),
        pltpu.SemaphoreType.DMA((NRB,)),
        pltpu.SemaphoreType.DMA((NRB,)),
        pltpu.SemaphoreType.DMA((NRB,)),
    ],
    compiler_params=pltpu.CompilerParams(use_tc_tiling_on_sc=False,
                                         needs_layout_passes=False),
)

_BLK = 2000


def _dense1(x, W, a2):
    return pl.pallas_call(
        _gat_dense_kernel,
        grid=(N // _BLK,),
        in_specs=[
            pl.BlockSpec((_BLK, D_IN), lambda i: (i, 0)),
            pl.BlockSpec((D_IN, HID), lambda i: (0, 0)),
            pl.BlockSpec((HID, 2), lambda i: (0, 0)),
        ],
        out_specs=[
            pl.BlockSpec((_BLK, DEXT), lambda i: (i, 0)),
            pl.BlockSpec((_BLK, 1), lambda i: (i, 0)),
        ],
        out_shape=[
            jax.ShapeDtypeStruct((N, DEXT), jnp.float32),
            jax.ShapeDtypeStruct((N, 1), jnp.float32),
        ],
    )(x, W, a2)


def _dense2(acc, b, W, a2):
    return pl.pallas_call(
        _gat_norm_dense_kernel,
        grid=(N // _BLK,),
        in_specs=[
            pl.BlockSpec((NC, _BLK, DEXT), lambda i: (0, i, 0)),
            pl.BlockSpec((1, HID), lambda i: (0, 0)),
            pl.BlockSpec((HID, HID), lambda i: (0, 0)),
            pl.BlockSpec((HID, 2), lambda i: (0, 0)),
        ],
        out_specs=[
            pl.BlockSpec((_BLK, DEXT), lambda i: (i, 0)),
            pl.BlockSpec((_BLK, 1), lambda i: (i, 0)),
        ],
        out_shape=[
            jax.ShapeDtypeStruct((N, DEXT), jnp.float32),
            jax.ShapeDtypeStruct((N, 1), jnp.float32),
        ],
    )(acc, b, W, a2)


def _finalize(acc, b, batch_row, fc_w, fc_b):
    return pl.pallas_call(
        _finalize_kernel,
        out_shape=jax.ShapeDtypeStruct((G, LAT), jnp.float32),
    )(acc, b, batch_row, fc_w, fc_b)


@jax.jit
def kernel(x, edge_index, batch, W1, a1_src, a1_dst, b1,
           W2, a2_src, a2_dst, b2, fc_w, fc_b):
    src = edge_index[0].reshape(NW * NCHUNK, CH)
    dst = edge_index[1].reshape(NW * NCHUNK, CH)
    zeros = jnp.zeros((N, DEXT), jnp.float32)

    a1 = jnp.stack([a1_src, a1_dst], axis=1)
    hext1, ad1 = _dense1(x, W1, a1)
    acc1 = _edge_pass(hext1, ad1.reshape(N), src, dst, zeros)

    a2 = jnp.stack([a2_src, a2_dst], axis=1)
    hext2, ad2 = _dense2(acc1, b1.reshape(1, HID), W2, a2)
    acc2 = _edge_pass(hext2, ad2.reshape(N), src, dst, zeros)

    return _finalize(acc2, b2.reshape(1, HID), batch.reshape(1, N),
                     fc_w, fc_b.reshape(1, LAT))


# speedup vs baseline: 1.0304x; 1.0013x over previous
import jax
import jax.numpy as jnp
from jax import lax
from jax.experimental import pallas as pl
from jax.experimental.pallas import tpu as pltpu
from jax.experimental.pallas import tpu_sc as plsc

N = 10000
E = 320000
D_IN = 128
HID = 128
LAT = 64
G = 64

DEXT = 144
NC = 2
NS = 16
NW = NC * NS
EPW = E // NW
CH = 80
NCHUNK = EPW // CH
ROWS_PER_TILE = N // NS


def _gat_dense_kernel(x_ref, w_ref, a_ref, hext_ref, ad_ref):
    h = jnp.dot(x_ref[...], w_ref[...], preferred_element_type=jnp.float32)
    al = jnp.dot(h, a_ref[...], preferred_element_type=jnp.float32)
    nrow = h.shape[0]
    hext_ref[...] = jnp.concatenate(
        [h,
         jnp.ones((nrow, 1), jnp.float32),
         al[:, 0:1],
         jnp.zeros((nrow, DEXT - HID - 2), jnp.float32)], axis=1)
    ad_ref[...] = al[:, 1:2]


def _gat_norm_dense_kernel(acc_ref, b_ref, w_ref, a_ref, hext_ref, ad_ref):
    accsum = acc_ref[0] + acc_ref[1]
    denom = accsum[:, HID:HID + 1]
    g = jax.nn.relu(accsum[:, :HID] / (denom + 1e-16) + b_ref[...])
    h = jnp.dot(g, w_ref[...], preferred_element_type=jnp.float32)
    al = jnp.dot(h, a_ref[...], preferred_element_type=jnp.float32)
    nrow = h.shape[0]
    hext_ref[...] = jnp.concatenate(
        [h,
         jnp.ones((nrow, 1), jnp.float32),
         al[:, 0:1],
         jnp.zeros((nrow, DEXT - HID - 2), jnp.float32)], axis=1)
    ad_ref[...] = al[:, 1:2]


def _finalize_kernel(acc_ref, b_ref, batch_ref, fcw_ref, fcb_ref, z_ref):
    accsum = acc_ref[0] + acc_ref[1]
    denom = accsum[:, HID:HID + 1]
    g = jax.nn.relu(accsum[:, :HID] / (denom + 1e-16) + b_ref[...])
    gid = lax.broadcasted_iota(jnp.int32, (G, N), 0)
    oh = (gid == batch_ref[...]).astype(jnp.float32)
    pooled = jnp.dot(oh, g, preferred_element_type=jnp.float32)
    counts = jnp.sum(oh, axis=1, keepdims=True)
    pooled = pooled / (counts + 1e-16)
    z_ref[...] = jnp.dot(pooled, fcw_ref[...],
                         preferred_element_type=jnp.float32) + fcb_ref[...]


NRB = 3
NIB = 5


def _edge_body(hext_hbm, ad_hbm, src_hbm, dst_hbm, zeros_hbm, out_hbm,
               acc_s, sidx, didx, adb, rows, wbuf, isems, gsems, asems, ssems):
    c = lax.axis_index("c")
    s = lax.axis_index("s")
    wid = c * NS + s
    rbase = wid * NCHUNK

    def start_idx(j):
        sl = j % NIB
        pltpu.async_copy(src_hbm.at[rbase + j], sidx.at[sl], isems.at[sl])
        pltpu.async_copy(dst_hbm.at[rbase + j], didx.at[sl], isems.at[sl])

    def wait_idx(j):
        sl = j % NIB
        pltpu.make_async_copy(src_hbm.at[rbase + j], sidx.at[sl],
                              isems.at[sl]).wait()
        pltpu.make_async_copy(dst_hbm.at[rbase + j], didx.at[sl],
                              isems.at[sl]).wait()

    def start_gather(j):
        sl, rb = j % NIB, j % NRB
        pltpu.async_copy(hext_hbm.at[sidx.at[sl]], rows.at[rb], gsems.at[rb])
        pltpu.async_copy(ad_hbm.at[didx.at[sl]], adb.at[rb], asems.at[rb])

    def wait_gather(j):
        sl, rb = j % NIB, j % NRB
        pltpu.make_async_copy(hext_hbm.at[sidx.at[sl]], rows.at[rb],
                              gsems.at[rb]).wait()
        pltpu.make_async_copy(ad_hbm.at[didx.at[sl]], adb.at[rb],
                              asems.at[rb]).wait()

    def start_scatter(j):
        sl, rb = j % NIB, j % NRB
        pltpu.async_copy(rows.at[rb], acc_s.at[didx.at[sl]], ssems.at[rb],
                         add=True)

    def wait_scatter(j):
        sl, rb = j % NIB, j % NRB
        pltpu.make_async_copy(rows.at[rb], acc_s.at[didx.at[sl]],
                              ssems.at[rb]).wait()

    def compute(j):
        rb = j % NRB
        for gi in range(CH // 16):
            rowi = lax.iota(jnp.int32, 16) + gi * 16
            asv = plsc.load_gather(rows.at[rb],
                                   [rowi, jnp.full((16,), HID + 1, jnp.int32)])
            adv = adb[rb, pl.ds(gi * 16, 16)]
            e = asv + adv
            e = jnp.where(e >= 0.0, e, 0.2 * e)
            wbuf[pl.ds(gi * 16, 16)] = jnp.exp(e)

        def scale(i, carry2):
            wbc = plsc.load_gather(wbuf, [jnp.full((16,), 0, jnp.int32) + i])
            for f in range(HID // 16):
                rows[rb, i, pl.ds(f * 16, 16)] = (
                    rows[rb, i, pl.ds(f * 16, 16)] * wbc)
            rows[rb, i, pl.ds(HID, 16)] = wbc
            return carry2
        lax.fori_loop(0, CH, scale, 0, unroll=4)

    start_idx(0)
    start_idx(1)
    wait_idx(0); start_gather(0); start_idx(2)
    wait_idx(1); start_gather(1); start_idx(3)
    pltpu.sync_copy(zeros_hbm.at[pl.ds(s * ROWS_PER_TILE, ROWS_PER_TILE)],
                    acc_s.at[pl.ds(s * ROWS_PER_TILE, ROWS_PER_TILE)])
    plsc.subcore_barrier()
    wait_gather(0); compute(0); start_scatter(0)
    wait_idx(2); start_gather(2); start_idx(4)
    wait_gather(1); compute(1); start_scatter(1)

    def steady(q, carry):
        for k in range(15):
            j = 2 + q * 15 + k
            wait_scatter(j - 2)
            wait_idx(j + 1)
            start_gather(j + 1)
            start_idx(j + 3)
            wait_gather(j)
            compute(j)
            start_scatter(j)
        return carry
    lax.fori_loop(0, (NCHUNK - 5) // 15, steady, 0)

    for j in range(NCHUNK - 3, NCHUNK):
        wait_scatter(j - 2)
        if j + 1 < NCHUNK:
            wait_idx(j + 1)
            start_gather(j + 1)
        wait_gather(j)
        compute(j)
        start_scatter(j)
    wait_scatter(NCHUNK - 2)
    wait_scatter(NCHUNK - 1)

    plsc.subcore_barrier()
    pltpu.sync_copy(acc_s.at[pl.ds(s * ROWS_PER_TILE, ROWS_PER_TILE)],
                    out_hbm.at[c, pl.ds(s * ROWS_PER_TILE, ROWS_PER_TILE)])


_edge_pass = pl.kernel(
    _edge_body,
    out_type=jax.ShapeDtypeStruct((NC, N, DEXT), jnp.float32),
    mesh=plsc.VectorSubcoreMesh(core_axis_name="c", subcore_axis_name="s"),
    scratch_types=[
        pltpu.VMEM_SHARED((N, DEXT), jnp.float32),
        pltpu.VMEM((NIB, CH), jnp.int32),
        pltpu.VMEM((NIB, CH), jnp.int32),
        pltpu.VMEM((NRB, CH), jnp.float32),
        pltpu.VMEM((NRB, CH, DEXT), jnp.float32),
        pltpu.VMEM((CH,), jnp.float32),
        pltpu.SemaphoreType.DMA((NIB,)),
        pltpu.SemaphoreType.DMA((NRB,)),
        pltpu.SemaphoreType.DMA((NRB,)),
        pltpu.SemaphoreType.DMA((NRB,)),
    ],
    compiler_params=pltpu.CompilerParams(use_tc_tiling_on_sc=False,
                                         needs_layout_passes=False),
)

_BLK = 2000


def _dense1(x, W, a2):
    return pl.pallas_call(
        _gat_dense_kernel,
        grid=(N // _BLK,),
        in_specs=[
            pl.BlockSpec((_BLK, D_IN), lambda i: (i, 0)),
            pl.BlockSpec((D_IN, HID), lambda i: (0, 0)),
            pl.BlockSpec((HID, 2), lambda i: (0, 0)),
        ],
        out_specs=[
            pl.BlockSpec((_BLK, DEXT), lambda i: (i, 0)),
            pl.BlockSpec((_BLK, 1), lambda i: (i, 0)),
        ],
        out_shape=[
            jax.ShapeDtypeStruct((N, DEXT), jnp.float32),
            jax.ShapeDtypeStruct((N, 1), jnp.float32),
        ],
    )(x, W, a2)


def _dense2(acc, b, W, a2):
    return pl.pallas_call(
        _gat_norm_dense_kernel,
        grid=(N // _BLK,),
        in_specs=[
            pl.BlockSpec((NC, _BLK, DEXT), lambda i: (0, i, 0)),
            pl.BlockSpec((1, HID), lambda i: (0, 0)),
            pl.BlockSpec((HID, HID), lambda i: (0, 0)),
            pl.BlockSpec((HID, 2), lambda i: (0, 0)),
        ],
        out_specs=[
            pl.BlockSpec((_BLK, DEXT), lambda i: (i, 0)),
            pl.BlockSpec((_BLK, 1), lambda i: (i, 0)),
        ],
        out_shape=[
            jax.ShapeDtypeStruct((N, DEXT), jnp.float32),
            jax.ShapeDtypeStruct((N, 1), jnp.float32),
        ],
    )(acc, b, W, a2)


def _finalize(acc, b, batch_row, fc_w, fc_b):
    return pl.pallas_call(
        _finalize_kernel,
        out_shape=jax.ShapeDtypeStruct((G, LAT), jnp.float32),
    )(acc, b, batch_row, fc_w, fc_b)


@jax.jit
def kernel(x, edge_index, batch, W1, a1_src, a1_dst, b1,
           W2, a2_src, a2_dst, b2, fc_w, fc_b):
    src = edge_index[0].reshape(NW * NCHUNK, CH)
    dst = edge_index[1].reshape(NW * NCHUNK, CH)
    zeros = jnp.zeros((N, DEXT), jnp.float32)

    a1 = jnp.stack([a1_src, a1_dst], axis=1)
    hext1, ad1 = _dense1(x, W1, a1)
    acc1 = _edge_pass(hext1, ad1.reshape(N), src, dst, zeros)

    a2 = jnp.stack([a2_src, a2_dst], axis=1)
    hext2, ad2 = _dense2(acc1, b1.reshape(1, HID), W2, a2)
    acc2 = _edge_pass(hext2, ad2.reshape(N), src, dst, zeros)

    return _finalize(acc2, b2.reshape(1, HID), batch.reshape(1, N),
                     fc_w, fc_b.reshape(1, LAT))


# speedup vs baseline: 1.1540x; 1.1199x over previous
import jax
import jax.numpy as jnp
from jax import lax
from jax.experimental import pallas as pl
from jax.experimental.pallas import tpu as pltpu
from jax.experimental.pallas import tpu_sc as plsc

N = 10000
E = 320000
D_IN = 128
HID = 128
LAT = 64
G = 64

NC = 2
NS = 16
NW = NC * NS
EPW = E // NW
CH = 80
NCHUNK = EPW // CH
ROWS_PER_TILE = N // NS


def _gat_dense_kernel(x_ref, w_ref, a_ref, h_ref, al_ref):
    h = jnp.dot(x_ref[...], w_ref[...], preferred_element_type=jnp.float32)
    h_ref[...] = h
    al_ref[...] = jnp.dot(h, a_ref[...], preferred_element_type=jnp.float32)


def _gat_norm_dense_kernel(acc_ref, den_ref, b_ref, w_ref, a_ref, h_ref,
                           al_ref):
    accsum = acc_ref[0] + acc_ref[1]
    denom = den_ref[0] + den_ref[1]
    g = jax.nn.relu(accsum / (denom + 1e-16) + b_ref[...])
    h = jnp.dot(g, w_ref[...], preferred_element_type=jnp.float32)
    h_ref[...] = h
    al_ref[...] = jnp.dot(h, a_ref[...], preferred_element_type=jnp.float32)


def _finalize_kernel(acc_ref, den_ref, b_ref, batch_ref, fcw_ref, fcb_ref,
                     z_ref):
    accsum = acc_ref[0] + acc_ref[1]
    denom = den_ref[0] + den_ref[1]
    g = jax.nn.relu(accsum / (denom + 1e-16) + b_ref[...])
    gid = lax.broadcasted_iota(jnp.int32, (G, N), 0)
    oh = (gid == batch_ref[...]).astype(jnp.float32)
    pooled = jnp.dot(oh, g, preferred_element_type=jnp.float32)
    counts = jnp.sum(oh, axis=1, keepdims=True)
    pooled = pooled / (counts + 1e-16)
    z_ref[...] = jnp.dot(pooled, fcw_ref[...],
                         preferred_element_type=jnp.float32) + fcb_ref[...]


NRB = 3
NIB = 5


def _edge_body(h_hbm, as_hbm, ad_hbm, src_hbm, dst_hbm, zeros_hbm, zd_hbm,
               acc_hbm, den_hbm,
               acc_s, den_s, as_v, sidx, didx, adb, rows, wbufs,
               isems, gsems, asems, ssems, dsems):
    c = lax.axis_index("c")
    s = lax.axis_index("s")
    wid = c * NS + s
    rbase = wid * NCHUNK

    def start_idx(j):
        sl = j % NIB
        pltpu.async_copy(src_hbm.at[rbase + j], sidx.at[sl], isems.at[sl])
        pltpu.async_copy(dst_hbm.at[rbase + j], didx.at[sl], isems.at[sl])

    def wait_idx(j):
        sl = j % NIB
        pltpu.make_async_copy(src_hbm.at[rbase + j], sidx.at[sl],
                              isems.at[sl]).wait()
        pltpu.make_async_copy(dst_hbm.at[rbase + j], didx.at[sl],
                              isems.at[sl]).wait()

    def start_gather(j):
        sl, rb = j % NIB, j % NRB
        pltpu.async_copy(h_hbm.at[sidx.at[sl]], rows.at[rb], gsems.at[rb])
        pltpu.async_copy(ad_hbm.at[didx.at[sl]], adb.at[rb], asems.at[rb])

    def wait_gather(j):
        sl, rb = j % NIB, j % NRB
        pltpu.make_async_copy(h_hbm.at[sidx.at[sl]], rows.at[rb],
                              gsems.at[rb]).wait()
        pltpu.make_async_copy(ad_hbm.at[didx.at[sl]], adb.at[rb],
                              asems.at[rb]).wait()

    def start_scatter(j):
        sl, rb = j % NIB, j % NRB
        pltpu.async_copy(rows.at[rb], acc_s.at[didx.at[sl]], ssems.at[rb],
                         add=True)
        pltpu.async_copy(wbufs.at[rb], den_s.at[didx.at[sl]], dsems.at[rb],
                         add=True)

    def wait_scatter(j):
        sl, rb = j % NIB, j % NRB
        pltpu.make_async_copy(rows.at[rb], acc_s.at[didx.at[sl]],
                              ssems.at[rb]).wait()
        pltpu.make_async_copy(wbufs.at[rb], den_s.at[didx.at[sl]],
                              dsems.at[rb]).wait()

    def compute(j):
        sl, rb = j % NIB, j % NRB
        for gi in range(CH // 16):
            srcv = sidx[sl, pl.ds(gi * 16, 16)]
            asv = plsc.load_gather(as_v, [srcv])
            adv = adb[rb, pl.ds(gi * 16, 16)]
            e = asv + adv
            e = jnp.where(e >= 0.0, e, 0.2 * e)
            wbufs[rb, pl.ds(gi * 16, 16)] = jnp.exp(e)

        def scale(i, carry2):
            wbc = plsc.load_gather(wbufs.at[rb],
                                   [jnp.full((16,), 0, jnp.int32) + i])
            for f in range(HID // 16):
                rows[rb, i, pl.ds(f * 16, 16)] = (
                    rows[rb, i, pl.ds(f * 16, 16)] * wbc)
            return carry2
        lax.fori_loop(0, CH, scale, 0, unroll=4)

    start_idx(0)
    start_idx(1)
    wait_idx(0); start_gather(0); start_idx(2)
    wait_idx(1); start_gather(1); start_idx(3)
    pltpu.sync_copy(zeros_hbm.at[pl.ds(s * ROWS_PER_TILE, ROWS_PER_TILE)],
                    acc_s.at[pl.ds(s * ROWS_PER_TILE, ROWS_PER_TILE)])
    pltpu.sync_copy(zd_hbm.at[pl.ds(s * 624, 624)],
                    den_s.at[pl.ds(s * 624, 624)])

    @pl.when(s == NS - 1)
    def _zero_tail():
        pltpu.sync_copy(zd_hbm.at[pl.ds(N - 16, 16)],
                        den_s.at[pl.ds(N - 16, 16)])
    pltpu.sync_copy(as_hbm, as_v)
    plsc.subcore_barrier()
    wait_gather(0); compute(0); start_scatter(0)
    wait_idx(2); start_gather(2); start_idx(4)
    wait_gather(1); compute(1); start_scatter(1)

    def steady(q, carry):
        for k in range(15):
            j = 2 + q * 15 + k
            wait_scatter(j - 2)
            wait_idx(j + 1)
            start_gather(j + 1)
            start_idx(j + 3)
            wait_gather(j)
            compute(j)
            start_scatter(j)
        return carry
    lax.fori_loop(0, (NCHUNK - 5) // 15, steady, 0)

    for j in range(NCHUNK - 3, NCHUNK):
        wait_scatter(j - 2)
        if j + 1 < NCHUNK:
            wait_idx(j + 1)
            start_gather(j + 1)
        wait_gather(j)
        compute(j)
        start_scatter(j)
    wait_scatter(NCHUNK - 2)
    wait_scatter(NCHUNK - 1)

    plsc.subcore_barrier()
    pltpu.sync_copy(acc_s.at[pl.ds(s * ROWS_PER_TILE, ROWS_PER_TILE)],
                    acc_hbm.at[c, pl.ds(s * ROWS_PER_TILE, ROWS_PER_TILE)])
    pltpu.sync_copy(den_s.at[pl.ds(s * 624, 624)],
                    den_hbm.at[c, pl.ds(s * 624, 624)])

    @pl.when(s == NS - 1)
    def _write_tail():
        pltpu.sync_copy(den_s.at[pl.ds(N - 16, 16)],
                        den_hbm.at[c, pl.ds(N - 16, 16)])


_edge_pass = pl.kernel(
    _edge_body,
    out_type=(jax.ShapeDtypeStruct((NC, N, HID), jnp.float32),
              jax.ShapeDtypeStruct((NC, N), jnp.float32)),
    mesh=plsc.VectorSubcoreMesh(core_axis_name="c", subcore_axis_name="s"),
    scratch_types=[
        pltpu.VMEM_SHARED((N, HID), jnp.float32),
        pltpu.VMEM_SHARED((N,), jnp.float32),
        pltpu.VMEM((N,), jnp.float32),
        pltpu.VMEM((NIB, CH), jnp.int32),
        pltpu.VMEM((NIB, CH), jnp.int32),
        pltpu.VMEM((NRB, CH), jnp.float32),
        pltpu.VMEM((NRB, CH, HID), jnp.float32),
        pltpu.VMEM((NRB, CH), jnp.float32),
        pltpu.SemaphoreType.DMA((NIB,)),
        pltpu.SemaphoreType.DMA((NRB,)),
        pltpu.SemaphoreType.DMA((NRB,)),
        pltpu.SemaphoreType.DMA((NRB,)),
        pltpu.SemaphoreType.DMA((NRB,)),
    ],
    compiler_params=pltpu.CompilerParams(use_tc_tiling_on_sc=False,
                                         needs_layout_passes=False),
)


_BLK = 2000


def _dense1(x, W, a2):
    return pl.pallas_call(
        _gat_dense_kernel,
        grid=(N // _BLK,),
        in_specs=[
            pl.BlockSpec((_BLK, D_IN), lambda i: (i, 0)),
            pl.BlockSpec((D_IN, HID), lambda i: (0, 0)),
            pl.BlockSpec((HID, 2), lambda i: (0, 0)),
        ],
        out_specs=[
            pl.BlockSpec((_BLK, HID), lambda i: (i, 0)),
            pl.BlockSpec((_BLK, 2), lambda i: (i, 0)),
        ],
        out_shape=[
            jax.ShapeDtypeStruct((N, HID), jnp.float32),
            jax.ShapeDtypeStruct((N, 2), jnp.float32),
        ],
    )(x, W, a2)


def _dense2(acc, den, b, W, a2):
    return pl.pallas_call(
        _gat_norm_dense_kernel,
        grid=(N // _BLK,),
        in_specs=[
            pl.BlockSpec((NC, _BLK, HID), lambda i: (0, i, 0)),
            pl.BlockSpec((NC, _BLK, 1), lambda i: (0, i, 0)),
            pl.BlockSpec((1, HID), lambda i: (0, 0)),
            pl.BlockSpec((HID, HID), lambda i: (0, 0)),
            pl.BlockSpec((HID, 2), lambda i: (0, 0)),
        ],
        out_specs=[
            pl.BlockSpec((_BLK, HID), lambda i: (i, 0)),
            pl.BlockSpec((_BLK, 2), lambda i: (i, 0)),
        ],
        out_shape=[
            jax.ShapeDtypeStruct((N, HID), jnp.float32),
            jax.ShapeDtypeStruct((N, 2), jnp.float32),
        ],
    )(acc, den, b, W, a2)


def _finalize(acc, den, b, batch_row, fc_w, fc_b):
    return pl.pallas_call(
        _finalize_kernel,
        out_shape=jax.ShapeDtypeStruct((G, LAT), jnp.float32),
    )(acc, den, b, batch_row, fc_w, fc_b)


@jax.jit
def kernel(x, edge_index, batch, W1, a1_src, a1_dst, b1,
           W2, a2_src, a2_dst, b2, fc_w, fc_b):
    src = edge_index[0].reshape(NW * NCHUNK, CH)
    dst = edge_index[1].reshape(NW * NCHUNK, CH)
    zeros = jnp.zeros((N, HID), jnp.float32)
    zd = jnp.zeros((N,), jnp.float32)

    a1 = jnp.stack([a1_src, a1_dst], axis=1)
    h1, al1 = _dense1(x, W1, a1)
    acc1, den1 = _edge_pass(h1, al1[:, 0], al1[:, 1], src, dst, zeros, zd)

    a2 = jnp.stack([a2_src, a2_dst], axis=1)
    h2, al2 = _dense2(acc1, den1.reshape(NC, N, 1), b1.reshape(1, HID), W2, a2)
    acc2, den2 = _edge_pass(h2, al2[:, 0], al2[:, 1], src, dst, zeros, zd)

    return _finalize(acc2, den2.reshape(NC, N, 1), b2.reshape(1, HID),
                     batch.reshape(1, N), fc_w, fc_b.reshape(1, LAT))
